# Initial kernel scaffold; baseline (speedup 1.0000x reference)
import jax
import jax.numpy as jnp
from jax.experimental import pallas as pl


def kernel(x, conv_bias, lower_bound1, q1):
    raise NotImplementedError("write your pallas kernel here")


# speedup vs baseline: 5.8642x; 5.8642x over previous
import jax
import jax.numpy as jnp
from jax.experimental import pallas as pl
from jax.experimental.pallas import tpu as pltpu

_B, _H, _W = 64, 512, 512


def _shift_right(a, k):
    return jnp.concatenate(
        [jnp.zeros((a.shape[0], k), jnp.float32), a[:, : a.shape[1] - k]], axis=1
    )


def _shift_left(a, k):
    return jnp.concatenate(
        [a[:, k:], jnp.zeros((a.shape[0], k), jnp.float32)], axis=1
    )


def _body(s_ref, x_ref, o_ref):
    bias = s_ref[0]
    lb = s_ref[1]
    q = s_ref[2]
    x = x_ref[0]
    x = x.astype(jnp.bfloat16).astype(jnp.float32)

    xr = x.reshape(_H // 2, 2, _W)
    even_rows = xr[:, 0, :]
    odd_rows = xr[:, 1, :]
    odd_shifted = jnp.concatenate(
        [jnp.zeros((1, _W), jnp.float32), odd_rows[: _H // 2 - 1, :]], axis=0
    )
    v = even_rows + odd_shifted

    convf = _shift_right(v, 1) + v + bias

    rneg = jnp.maximum(-convf, 0.0)

    hmax = jnp.maximum(rneg, _shift_left(rneg, 2))
    hsum = convf + _shift_left(convf, 2)

    hmax_r = hmax.reshape(128, 2, _W)
    m = jnp.maximum(hmax_r[:, 0, :], hmax_r[:, 1, :])
    hsum_r = hsum.reshape(128, 2, _W)
    mean = (hsum_r[:, 0, :] + hsum_r[:, 1, :]) * 0.25

    neg_m = -m
    cond = (neg_m < lb) & ((mean / neg_m) > (q / lb))
    val = jnp.where(cond, jnp.float32(0.0), jnp.float32(1.0))

    lane = jax.lax.broadcasted_iota(jnp.int32, (128, _W), 1)
    w0 = jnp.where(lane % 4 == 0, val, 0.0)
    w = w0 + _shift_right(w0, 1) + _shift_right(w0, 2) + _shift_right(w0, 3)

    up = jnp.broadcast_to(w.reshape(128, 1, _W), (128, 4, _W)).reshape(_H, _W)
    o_ref[0] = up


def kernel(x, conv_bias, lower_bound1, q1):
    xs = x.reshape(_B, _H, _W)
    scalars = jnp.stack(
        [conv_bias.reshape(()), lower_bound1.reshape(()), q1.reshape(())]
    ).astype(jnp.float32)
    out = pl.pallas_call(
        _body,
        grid=(_B,),
        in_specs=[
            pl.BlockSpec(memory_space=pltpu.SMEM),
            pl.BlockSpec((1, _H, _W), lambda b: (b, 0, 0)),
        ],
        out_specs=pl.BlockSpec((1, _H, _W), lambda b: (b, 0, 0)),
        out_shape=jax.ShapeDtypeStruct((_B, _H, _W), jnp.float32),
    )(scalars, xs)
    return out.reshape(_B, 1, _H, _W)


# speedup vs baseline: 6.8078x; 1.1609x over previous
import jax
import jax.numpy as jnp
from jax.experimental import pallas as pl
from jax.experimental.pallas import tpu as pltpu

_B, _H, _W = 64, 512, 512


def _shift_right(a, k):
    return jnp.concatenate(
        [jnp.zeros((a.shape[0], k), jnp.float32), a[:, : a.shape[1] - k]], axis=1
    )


def _shift_left(a, k):
    return jnp.concatenate(
        [a[:, k:], jnp.zeros((a.shape[0], k), jnp.float32)], axis=1
    )


def _body(s_ref, a_ref, x_ref, o_ref):
    bias = s_ref[0]
    lb = s_ref[1]
    q = s_ref[2]
    x = x_ref[0]

    xb = x.astype(jnp.bfloat16)
    v = jnp.dot(
        a_ref[...], xb, preferred_element_type=jnp.float32
    )

    convf = _shift_right(v, 1) + v + bias

    rneg = jnp.maximum(-convf, 0.0)

    hmax = jnp.maximum(rneg, _shift_left(rneg, 2))
    hsum = convf + _shift_left(convf, 2)

    hmax_r = hmax.reshape(128, 2, _W)
    m = jnp.maximum(hmax_r[:, 0, :], hmax_r[:, 1, :])
    hsum_r = hsum.reshape(128, 2, _W)
    mean = (hsum_r[:, 0, :] + hsum_r[:, 1, :]) * 0.25

    neg_m = -m
    cond = (neg_m < lb) & ((mean / neg_m) > (q / lb))
    val = jnp.where(cond, jnp.float32(0.0), jnp.float32(1.0))

    lane = jax.lax.broadcasted_iota(jnp.int32, (128, _W), 1)
    w0 = jnp.where(lane % 4 == 0, val, 0.0)
    w = w0 + _shift_right(w0, 1) + _shift_right(w0, 2) + _shift_right(w0, 3)

    up = jnp.broadcast_to(w.reshape(128, 1, _W), (128, 4, _W)).reshape(_H, _W)
    o_ref[0] = up


def kernel(x, conv_bias, lower_bound1, q1):
    xs = x.reshape(_B, _H, _W)
    scalars = jnp.stack(
        [conv_bias.reshape(()), lower_bound1.reshape(()), q1.reshape(())]
    ).astype(jnp.float32)
    rows = jnp.arange(_H // 2)[:, None]
    cols = jnp.arange(_H)[None, :]
    sel = ((cols == 2 * rows) | (cols == 2 * rows - 1)).astype(jnp.bfloat16)
    out = pl.pallas_call(
        _body,
        grid=(_B,),
        in_specs=[
            pl.BlockSpec(memory_space=pltpu.SMEM),
            pl.BlockSpec((_H // 2, _H), lambda b: (0, 0)),
            pl.BlockSpec((1, _H, _W), lambda b: (b, 0, 0)),
        ],
        out_specs=pl.BlockSpec((1, _H, _W), lambda b: (b, 0, 0)),
        out_shape=jax.ShapeDtypeStruct((_B, _H, _W), jnp.float32),
    )(scalars, sel, xs)
    return out.reshape(_B, 1, _H, _W)


# speedup vs baseline: 8.1581x; 1.1984x over previous
import jax
import jax.numpy as jnp
from jax.experimental import pallas as pl
from jax.experimental.pallas import tpu as pltpu

_B, _H, _W = 64, 512, 512


def _shift_right(a, k):
    return jnp.concatenate(
        [jnp.zeros((a.shape[0], k), jnp.float32), a[:, : a.shape[1] - k]], axis=1
    )


def _shift_left(a, k):
    return jnp.concatenate(
        [a[:, k:], jnp.zeros((a.shape[0], k), jnp.float32)], axis=1
    )


_BLK = 2


def _body(s_ref, a_ref, x_ref, o_ref):
    for k in range(_BLK):
        _one_image(s_ref, a_ref, x_ref, o_ref, k)


def _one_image(s_ref, a_ref, x_ref, o_ref, k):
    bias = s_ref[0]
    lb = s_ref[1]
    q = s_ref[2]
    x = x_ref[k]

    xb = x.astype(jnp.bfloat16)
    v = jnp.dot(
        a_ref[...], xb, preferred_element_type=jnp.float32
    )

    convf = _shift_right(v, 1) + v + bias

    rneg = jnp.maximum(-convf, 0.0)

    hmax = jnp.maximum(rneg, _shift_left(rneg, 2))
    hsum = convf + _shift_left(convf, 2)

    hmax_r = hmax.reshape(128, 2, _W)
    m = jnp.maximum(hmax_r[:, 0, :], hmax_r[:, 1, :])
    hsum_r = hsum.reshape(128, 2, _W)
    mean = (hsum_r[:, 0, :] + hsum_r[:, 1, :]) * 0.25

    neg_m = -m
    cond = (neg_m < lb) & ((mean / neg_m) > (q / lb))
    val = jnp.where(cond, jnp.float32(0.0), jnp.float32(1.0))

    lane = jax.lax.broadcasted_iota(jnp.int32, (128, _W), 1)
    w0 = jnp.where(lane % 4 == 0, val, 0.0)
    w = w0 + _shift_right(w0, 1) + _shift_right(w0, 2) + _shift_right(w0, 3)

    up = jnp.broadcast_to(w.reshape(128, 1, _W), (128, 4, _W)).reshape(_H, _W)
    o_ref[k] = up


def kernel(x, conv_bias, lower_bound1, q1):
    xs = x.reshape(_B, _H, _W)
    scalars = jnp.stack(
        [conv_bias.reshape(()), lower_bound1.reshape(()), q1.reshape(())]
    ).astype(jnp.float32)
    rows = jnp.arange(_H // 2)[:, None]
    cols = jnp.arange(_H)[None, :]
    sel = ((cols == 2 * rows) | (cols == 2 * rows - 1)).astype(jnp.bfloat16)
    out = pl.pallas_call(
        _body,
        grid=(_B // _BLK,),
        in_specs=[
            pl.BlockSpec(memory_space=pltpu.SMEM),
            pl.BlockSpec((_H // 2, _H), lambda b: (0, 0)),
            pl.BlockSpec((_BLK, _H, _W), lambda b: (b, 0, 0)),
        ],
        out_specs=pl.BlockSpec((_BLK, _H, _W), lambda b: (b, 0, 0)),
        out_shape=jax.ShapeDtypeStruct((_B, _H, _W), jnp.float32),
    )(scalars, sel, xs)
    return out.reshape(_B, 1, _H, _W)


# speedup vs baseline: 8.4760x; 1.0390x over previous
import jax
import jax.numpy as jnp
from jax.experimental import pallas as pl
from jax.experimental.pallas import tpu as pltpu

_B, _H, _W = 64, 512, 512


def _shift_right(a, k):
    return jnp.concatenate(
        [jnp.zeros((a.shape[0], k), jnp.float32), a[:, : a.shape[1] - k]], axis=1
    )


def _shift_left(a, k):
    return jnp.concatenate(
        [a[:, k:], jnp.zeros((a.shape[0], k), jnp.float32)], axis=1
    )


_BLK = 4


def _body(s_ref, a_ref, x_ref, o_ref):
    for k in range(_BLK):
        _one_image(s_ref, a_ref, x_ref, o_ref, k)


def _one_image(s_ref, a_ref, x_ref, o_ref, k):
    bias = s_ref[0]
    lb = s_ref[1]
    q = s_ref[2]
    x = x_ref[k]

    xb = x.astype(jnp.bfloat16)
    v = jnp.dot(
        a_ref[...], xb, preferred_element_type=jnp.float32
    )

    convf = _shift_right(v, 1) + v + bias

    rneg = jnp.maximum(-convf, 0.0)

    hmax = jnp.maximum(rneg, _shift_left(rneg, 2))
    hsum = convf + _shift_left(convf, 2)

    hmax_r = hmax.reshape(128, 2, _W)
    m = jnp.maximum(hmax_r[:, 0, :], hmax_r[:, 1, :])
    hsum_r = hsum.reshape(128, 2, _W)
    mean = (hsum_r[:, 0, :] + hsum_r[:, 1, :]) * 0.25

    neg_m = -m
    cond = (neg_m < lb) & ((mean / neg_m) > (q / lb))
    val = jnp.where(cond, jnp.float32(0.0), jnp.float32(1.0))

    lane = jax.lax.broadcasted_iota(jnp.int32, (128, _W), 1)
    w0 = jnp.where(lane % 4 == 0, val, 0.0)
    w = w0 + _shift_right(w0, 1) + _shift_right(w0, 2) + _shift_right(w0, 3)

    up = jnp.broadcast_to(w.reshape(128, 1, _W), (128, 4, _W)).reshape(_H, _W)
    o_ref[k] = up


def kernel(x, conv_bias, lower_bound1, q1):
    xs = x.reshape(_B, _H, _W)
    scalars = jnp.stack(
        [conv_bias.reshape(()), lower_bound1.reshape(()), q1.reshape(())]
    ).astype(jnp.float32)
    rows = jnp.arange(_H // 2)[:, None]
    cols = jnp.arange(_H)[None, :]
    sel = ((cols == 2 * rows) | (cols == 2 * rows - 1)).astype(jnp.bfloat16)
    out = pl.pallas_call(
        _body,
        grid=(_B // _BLK,),
        in_specs=[
            pl.BlockSpec(memory_space=pltpu.SMEM),
            pl.BlockSpec((_H // 2, _H), lambda b: (0, 0)),
            pl.BlockSpec((_BLK, _H, _W), lambda b: (b, 0, 0)),
        ],
        out_specs=pl.BlockSpec((_BLK, _H, _W), lambda b: (b, 0, 0)),
        out_shape=jax.ShapeDtypeStruct((_B, _H, _W), jnp.float32),
    )(scalars, sel, xs)
    return out.reshape(_B, 1, _H, _W)
